# baseline (speedup 1.0000x reference)
import jax
import jax.numpy as jnp
from jax import lax
from jax.experimental import pallas as pl
from jax.experimental.pallas import tpu as pltpu
from jax.experimental.pallas import tpu_sc as plsc

_N, _F, _T = 8192, 2048, 4
_NC, _NS, _L = 2, 16, 16
_NW = _NC * _NS
_CHUNK = 16384
_EPW = (_N * _F) // _NW
_NCHUNK = _EPW // _CHUNK
_CB = _F // _L
_ROWS = _CHUNK // _F


def _sc_body(x_hbm, alpha_hbm, logits_hbm, out_hbm,
             lg, av, cf, xb0, xb1, ob0, ob1, is0, is1, os0, os1):
    wid = lax.axis_index("s") * _NC + lax.axis_index("c")

    pltpu.sync_copy(logits_hbm, lg)
    pltpu.sync_copy(alpha_hbm, av)

    def coef_body(cb, carry):
        sl = pl.ds(cb * _L, _L)
        l0, l1, l2, l3 = lg[0, sl], lg[1, sl], lg[2, sl], lg[3, sl]
        m = jnp.maximum(jnp.maximum(l0, l1), jnp.maximum(l2, l3))
        e0 = jnp.exp(l0 - m)
        e1 = jnp.exp(l1 - m)
        e2 = jnp.exp(l2 - m)
        e3 = jnp.exp(l3 - m)
        r = av[sl] / (e0 + e1 + e2 + e3)
        cf[0, sl] = e0 * r
        cf[1, sl] = e1 * r
        cf[2, sl] = e2 * r
        cf[3, sl] = e3 * r
        return carry

    lax.fori_loop(0, _CB, coef_body, 0)

    base = wid * _EPW

    def _in_slice(k):
        return x_hbm.at[pl.ds(base + k * _CHUNK, _CHUNK)]

    def _out_slice(k):
        return out_hbm.at[pl.ds(base + k * _CHUNK, _CHUNK)]

    def compute(xbuf, obuf):
        @plsc.parallel_loop(0, _CB, step=1, unroll=2)
        def cb_body(cb):
            sl = pl.ds(cb * _L, _L)
            c0, c1, cq, cs = cf[0, sl], cf[1, sl], cf[2, sl], cf[3, sl]
            for r in range(_ROWS):
                i = r * _F + cb * _L
                x = xbuf[pl.ds(i, _L)]
                u = jnp.minimum(jnp.exp(-x), 1e9)
                u2 = u * u
                a1 = 1.0 + u
                a2 = 1.0 + u2
                d = 1.0 / (a1 * a2)
                num = c1 * a1 * (1.0 - u2) + cs * a2
                obuf[pl.ds(i, _L)] = x * (c0 + cq * x) + num * d

    pltpu.async_copy(_in_slice(0), xb0, is0)

    def pair_body(p, carry):
        k0 = 2 * p
        k1 = k0 + 1
        pltpu.async_copy(_in_slice(k1), xb1, is1)
        pltpu.make_async_copy(_in_slice(k0), xb0, is0).wait()

        @pl.when(p >= 1)
        def _():
            pltpu.make_async_copy(ob0, _out_slice(k0 - 2), os0).wait()

        compute(xb0, ob0)
        pltpu.async_copy(ob0, _out_slice(k0), os0)

        @pl.when(p + 1 < _NCHUNK // 2)
        def _():
            pltpu.async_copy(_in_slice(k0 + 2), xb0, is0)

        pltpu.make_async_copy(_in_slice(k1), xb1, is1).wait()

        @pl.when(p >= 1)
        def _():
            pltpu.make_async_copy(ob1, _out_slice(k1 - 2), os1).wait()

        compute(xb1, ob1)
        pltpu.async_copy(ob1, _out_slice(k1), os1)
        return carry

    lax.fori_loop(0, _NCHUNK // 2, pair_body, 0)
    pltpu.make_async_copy(ob0, _out_slice(_NCHUNK - 2), os0).wait()
    pltpu.make_async_copy(ob1, _out_slice(_NCHUNK - 1), os1).wait()


def kernel(X, alpha, tf_prob_logits):
    n, f = X.shape
    xf = X.reshape(-1)
    logits_t = tf_prob_logits.T

    mesh = plsc.VectorSubcoreMesh(core_axis_name="c", subcore_axis_name="s")
    run = pl.kernel(
        _sc_body,
        mesh=mesh,
        out_type=jax.ShapeDtypeStruct((n * f,), X.dtype),
        scratch_types=[
            pltpu.VMEM((_T, _F), jnp.float32),
            pltpu.VMEM((_F,), jnp.float32),
            pltpu.VMEM((_T, _F), jnp.float32),
            pltpu.VMEM((_CHUNK,), jnp.float32),
            pltpu.VMEM((_CHUNK,), jnp.float32),
            pltpu.VMEM((_CHUNK,), jnp.float32),
            pltpu.VMEM((_CHUNK,), jnp.float32),
            pltpu.SemaphoreType.DMA,
            pltpu.SemaphoreType.DMA,
            pltpu.SemaphoreType.DMA,
            pltpu.SemaphoreType.DMA,
        ],
    )
    return run(xf, alpha, logits_t).reshape(n, f)


# speedup vs baseline: 1.2148x; 1.2148x over previous
import jax
import jax.numpy as jnp
from jax import lax
from jax.experimental import pallas as pl
from jax.experimental.pallas import tpu as pltpu
from jax.experimental.pallas import tpu_sc as plsc

_N, _F, _T = 8192, 2048, 4
_NC, _NS, _L = 2, 16, 16
_NW = _NC * _NS
_CHUNK = 16384
_EPW = (_N * _F) // _NW
_NCHUNK = _EPW // _CHUNK
_CB = _F // _L
_ROWS = _CHUNK // _F


def _sc_body(x_hbm, alpha_hbm, logits_hbm, out_hbm,
             lg, av, cf, xb0, xb1, ob0, ob1, is0, is1, os0, os1):
    wid = lax.axis_index("s") * _NC + lax.axis_index("c")

    pltpu.sync_copy(logits_hbm, lg)
    pltpu.sync_copy(alpha_hbm, av)

    def coef_body(cb, carry):
        sl = pl.ds(cb * _L, _L)
        l0, l1, l2, l3 = lg[0, sl], lg[1, sl], lg[2, sl], lg[3, sl]
        m = jnp.maximum(jnp.maximum(l0, l1), jnp.maximum(l2, l3))
        e0 = jnp.exp(l0 - m)
        e1 = jnp.exp(l1 - m)
        e2 = jnp.exp(l2 - m)
        e3 = jnp.exp(l3 - m)
        r = av[sl] / (e0 + e1 + e2 + e3)
        cf[0, sl] = e0 * r
        cf[1, sl] = e1 * r
        cf[2, sl] = e2 * r
        cf[3, sl] = e3 * r
        return carry

    lax.fori_loop(0, _CB, coef_body, 0)

    base = wid * _EPW

    def _in_slice(k):
        return x_hbm.at[pl.ds(base + k * _CHUNK, _CHUNK)]

    def _out_slice(k):
        return out_hbm.at[pl.ds(base + k * _CHUNK, _CHUNK)]

    def compute(xbuf, obuf):
        @plsc.parallel_loop(0, _CB, step=1, unroll=1)
        def cb_body(cb):
            sl = pl.ds(cb * _L, _L)
            c0, c1, cq, cs = cf[0, sl], cf[1, sl], cf[2, sl], cf[3, sl]
            for r in range(_ROWS):
                i = r * _F + cb * _L
                x = xbuf[pl.ds(i, _L)]
                u = jnp.minimum(jnp.exp(-x), 1e9)
                u2 = u * u
                a1 = 1.0 + u
                a2 = 1.0 + u2
                d = 1.0 / (a1 * a2)
                num = c1 * a1 * (1.0 - u2) + cs * a2
                obuf[pl.ds(i, _L)] = x * (c0 + cq * x) + num * d

    pltpu.async_copy(_in_slice(0), xb0, is0)

    def pair_body(p, carry):
        k0 = 2 * p
        k1 = k0 + 1
        pltpu.async_copy(_in_slice(k1), xb1, is1)
        pltpu.make_async_copy(_in_slice(k0), xb0, is0).wait()

        @pl.when(p >= 1)
        def _():
            pltpu.make_async_copy(ob0, _out_slice(k0 - 2), os0).wait()

        compute(xb0, ob0)
        pltpu.async_copy(ob0, _out_slice(k0), os0)

        @pl.when(p + 1 < _NCHUNK // 2)
        def _():
            pltpu.async_copy(_in_slice(k0 + 2), xb0, is0)

        pltpu.make_async_copy(_in_slice(k1), xb1, is1).wait()

        @pl.when(p >= 1)
        def _():
            pltpu.make_async_copy(ob1, _out_slice(k1 - 2), os1).wait()

        compute(xb1, ob1)
        pltpu.async_copy(ob1, _out_slice(k1), os1)
        return carry

    lax.fori_loop(0, _NCHUNK // 2, pair_body, 0)
    pltpu.make_async_copy(ob0, _out_slice(_NCHUNK - 2), os0).wait()
    pltpu.make_async_copy(ob1, _out_slice(_NCHUNK - 1), os1).wait()


def kernel(X, alpha, tf_prob_logits):
    n, f = X.shape
    xf = X.reshape(-1)
    logits_t = tf_prob_logits.T

    mesh = plsc.VectorSubcoreMesh(core_axis_name="c", subcore_axis_name="s")
    run = pl.kernel(
        _sc_body,
        mesh=mesh,
        out_type=jax.ShapeDtypeStruct((n * f,), X.dtype),
        scratch_types=[
            pltpu.VMEM((_T, _F), jnp.float32),
            pltpu.VMEM((_F,), jnp.float32),
            pltpu.VMEM((_T, _F), jnp.float32),
            pltpu.VMEM((_CHUNK,), jnp.float32),
            pltpu.VMEM((_CHUNK,), jnp.float32),
            pltpu.VMEM((_CHUNK,), jnp.float32),
            pltpu.VMEM((_CHUNK,), jnp.float32),
            pltpu.SemaphoreType.DMA,
            pltpu.SemaphoreType.DMA,
            pltpu.SemaphoreType.DMA,
            pltpu.SemaphoreType.DMA,
        ],
    )
    return run(xf, alpha, logits_t).reshape(n, f)


# speedup vs baseline: 1.2155x; 1.0006x over previous
import jax
import jax.numpy as jnp
from jax import lax
from jax.experimental import pallas as pl
from jax.experimental.pallas import tpu as pltpu
from jax.experimental.pallas import tpu_sc as plsc

_N, _F, _T = 8192, 2048, 4
_NC, _NS, _L = 2, 16, 16
_NW = _NC * _NS
_CHUNK = 16384
_EPW = (_N * _F) // _NW
_NCHUNK = _EPW // _CHUNK
_CB = _F // _L
_ROWS = _CHUNK // _F


def _sc_body(x_hbm, alpha_hbm, logits_hbm, out_hbm,
             lg, av, cf, xb0, xb1, ob0, ob1, is0, is1, os0, os1):
    wid = lax.axis_index("s") * _NC + lax.axis_index("c")

    pltpu.sync_copy(logits_hbm, lg)
    pltpu.sync_copy(alpha_hbm, av)

    def coef_body(cb, carry):
        sl = pl.ds(cb * _L, _L)
        l0, l1, l2, l3 = lg[0, sl], lg[1, sl], lg[2, sl], lg[3, sl]
        m = jnp.maximum(jnp.maximum(l0, l1), jnp.maximum(l2, l3))
        e0 = jnp.exp(l0 - m)
        e1 = jnp.exp(l1 - m)
        e2 = jnp.exp(l2 - m)
        e3 = jnp.exp(l3 - m)
        r = av[sl] / (e0 + e1 + e2 + e3)
        cf[0, sl] = e0 * r
        cf[1, sl] = e1 * r
        cf[2, sl] = e2 * r
        cf[3, sl] = e3 * r
        return carry

    lax.fori_loop(0, _CB, coef_body, 0)

    base = wid * _EPW

    def _in_slice(k):
        return x_hbm.at[pl.ds(base + k * _CHUNK, _CHUNK)]

    def _out_slice(k):
        return out_hbm.at[pl.ds(base + k * _CHUNK, _CHUNK)]

    def compute(xbuf, obuf):
        @plsc.parallel_loop(0, _CHUNK // _L, step=1, unroll=8)
        def v_body(j):
            sl = pl.ds(jnp.bitwise_and(j, _CB - 1) * _L, _L)
            c0, c1, cq, cs = cf[0, sl], cf[1, sl], cf[2, sl], cf[3, sl]
            x = xbuf[pl.ds(j * _L, _L)]
            u = jnp.minimum(jnp.exp(-x), 1e9)
            u2 = u * u
            a1 = 1.0 + u
            a2 = 1.0 + u2
            d = 1.0 / (a1 * a2)
            num = c1 * a1 * (1.0 - u2) + cs * a2
            obuf[pl.ds(j * _L, _L)] = x * (c0 + cq * x) + num * d

    pltpu.async_copy(_in_slice(0), xb0, is0)

    def pair_body(p, carry):
        k0 = 2 * p
        k1 = k0 + 1
        pltpu.async_copy(_in_slice(k1), xb1, is1)
        pltpu.make_async_copy(_in_slice(k0), xb0, is0).wait()

        @pl.when(p >= 1)
        def _():
            pltpu.make_async_copy(ob0, _out_slice(k0 - 2), os0).wait()

        compute(xb0, ob0)
        pltpu.async_copy(ob0, _out_slice(k0), os0)

        @pl.when(p + 1 < _NCHUNK // 2)
        def _():
            pltpu.async_copy(_in_slice(k0 + 2), xb0, is0)

        pltpu.make_async_copy(_in_slice(k1), xb1, is1).wait()

        @pl.when(p >= 1)
        def _():
            pltpu.make_async_copy(ob1, _out_slice(k1 - 2), os1).wait()

        compute(xb1, ob1)
        pltpu.async_copy(ob1, _out_slice(k1), os1)
        return carry

    lax.fori_loop(0, _NCHUNK // 2, pair_body, 0)
    pltpu.make_async_copy(ob0, _out_slice(_NCHUNK - 2), os0).wait()
    pltpu.make_async_copy(ob1, _out_slice(_NCHUNK - 1), os1).wait()


def kernel(X, alpha, tf_prob_logits):
    n, f = X.shape
    xf = X.reshape(-1)
    logits_t = tf_prob_logits.T

    mesh = plsc.VectorSubcoreMesh(core_axis_name="c", subcore_axis_name="s")
    run = pl.kernel(
        _sc_body,
        mesh=mesh,
        out_type=jax.ShapeDtypeStruct((n * f,), X.dtype),
        scratch_types=[
            pltpu.VMEM((_T, _F), jnp.float32),
            pltpu.VMEM((_F,), jnp.float32),
            pltpu.VMEM((_T, _F), jnp.float32),
            pltpu.VMEM((_CHUNK,), jnp.float32),
            pltpu.VMEM((_CHUNK,), jnp.float32),
            pltpu.VMEM((_CHUNK,), jnp.float32),
            pltpu.VMEM((_CHUNK,), jnp.float32),
            pltpu.SemaphoreType.DMA,
            pltpu.SemaphoreType.DMA,
            pltpu.SemaphoreType.DMA,
            pltpu.SemaphoreType.DMA,
        ],
    )
    return run(xf, alpha, logits_t).reshape(n, f)


# speedup vs baseline: 2.1429x; 1.7630x over previous
import jax
import jax.numpy as jnp
from jax import lax
from jax.experimental import pallas as pl
from jax.experimental.pallas import tpu as pltpu
from jax.experimental.pallas import tpu_sc as plsc

_N, _F, _T = 8192, 2048, 4
_NC, _NS, _L = 2, 16, 16
_NW = _NC * _NS
_RPW = _N // _NW
_ROWS = 8
_NCHUNK = _RPW // _ROWS
_CB = _F // _L


def _sc_body(x_hbm, alpha_hbm, logits_hbm, out_hbm,
             lg, av, cf, xb0, xb1, ob0, ob1, is0, is1, os0, os1):
    wid = lax.axis_index("s") * _NC + lax.axis_index("c")

    pltpu.sync_copy(logits_hbm, lg)
    pltpu.sync_copy(alpha_hbm, av)

    def coef_body(cb, carry):
        sl = pl.ds(cb * _L, _L)
        l0, l1, l2, l3 = lg[0, sl], lg[1, sl], lg[2, sl], lg[3, sl]
        m = jnp.maximum(jnp.maximum(l0, l1), jnp.maximum(l2, l3))
        e0 = jnp.exp(l0 - m)
        e1 = jnp.exp(l1 - m)
        e2 = jnp.exp(l2 - m)
        e3 = jnp.exp(l3 - m)
        r = av[sl] / (e0 + e1 + e2 + e3)
        cf[0, sl] = e0 * r
        cf[1, sl] = e1 * r
        cf[2, sl] = e2 * r
        cf[3, sl] = e3 * r
        return carry

    lax.fori_loop(0, _CB, coef_body, 0)

    row0 = wid * _RPW

    def _in_slice(k):
        return x_hbm.at[pl.ds(row0 + k * _ROWS, _ROWS)]

    def _out_slice(k):
        return out_hbm.at[pl.ds(row0 + k * _ROWS, _ROWS)]

    def compute(xbuf, obuf):
        @plsc.parallel_loop(0, _CB, step=1, unroll=1)
        def cb_body(cb):
            sl = pl.ds(cb * _L, _L)
            c0, c1, cq, cs = cf[0, sl], cf[1, sl], cf[2, sl], cf[3, sl]
            for r in range(_ROWS):
                x = xbuf[r, sl]
                u = jnp.minimum(jnp.exp(-x), 1e9)
                u2 = u * u
                a1 = 1.0 + u
                a2 = 1.0 + u2
                d = 1.0 / (a1 * a2)
                num = c1 * a1 * (1.0 - u2) + cs * a2
                obuf[r, sl] = x * (c0 + cq * x) + num * d

    pltpu.async_copy(_in_slice(0), xb0, is0)

    def pair_body(p, carry):
        k0 = 2 * p
        k1 = k0 + 1
        pltpu.async_copy(_in_slice(k1), xb1, is1)
        pltpu.make_async_copy(_in_slice(k0), xb0, is0).wait()

        @pl.when(p >= 1)
        def _():
            pltpu.make_async_copy(ob0, _out_slice(k0 - 2), os0).wait()

        compute(xb0, ob0)
        pltpu.async_copy(ob0, _out_slice(k0), os0)

        @pl.when(p + 1 < _NCHUNK // 2)
        def _():
            pltpu.async_copy(_in_slice(k0 + 2), xb0, is0)

        pltpu.make_async_copy(_in_slice(k1), xb1, is1).wait()

        @pl.when(p >= 1)
        def _():
            pltpu.make_async_copy(ob1, _out_slice(k1 - 2), os1).wait()

        compute(xb1, ob1)
        pltpu.async_copy(ob1, _out_slice(k1), os1)
        return carry

    lax.fori_loop(0, _NCHUNK // 2, pair_body, 0)
    pltpu.make_async_copy(ob0, _out_slice(_NCHUNK - 2), os0).wait()
    pltpu.make_async_copy(ob1, _out_slice(_NCHUNK - 1), os1).wait()


def kernel(X, alpha, tf_prob_logits):
    n, f = X.shape
    logits_t = tf_prob_logits.T

    mesh = plsc.VectorSubcoreMesh(core_axis_name="c", subcore_axis_name="s")
    run = pl.kernel(
        _sc_body,
        mesh=mesh,
        out_type=jax.ShapeDtypeStruct((n, f), X.dtype),
        scratch_types=[
            pltpu.VMEM((_T, _F), jnp.float32),
            pltpu.VMEM((_F,), jnp.float32),
            pltpu.VMEM((_T, _F), jnp.float32),
            pltpu.VMEM((_ROWS, _F), jnp.float32),
            pltpu.VMEM((_ROWS, _F), jnp.float32),
            pltpu.VMEM((_ROWS, _F), jnp.float32),
            pltpu.VMEM((_ROWS, _F), jnp.float32),
            pltpu.SemaphoreType.DMA,
            pltpu.SemaphoreType.DMA,
            pltpu.SemaphoreType.DMA,
            pltpu.SemaphoreType.DMA,
        ],
    )
    return run(X, alpha, logits_t)


# speedup vs baseline: 2.1595x; 1.0078x over previous
import jax
import jax.numpy as jnp
from jax import lax
from jax.experimental import pallas as pl
from jax.experimental.pallas import tpu as pltpu
from jax.experimental.pallas import tpu_sc as plsc

_N, _F, _T = 8192, 2048, 4
_NC, _NS, _L = 2, 16, 16
_NW = _NC * _NS
_RPW = _N // _NW
_ROWS = 8
_NCHUNK = _RPW // _ROWS
_CB = _F // _L


def _sc_body(x_hbm, alpha_hbm, logits_hbm, out_hbm,
             lg, av, cf, xb0, xb1, ob0, ob1, is0, is1, os0, os1):
    wid = lax.axis_index("s") * _NC + lax.axis_index("c")
    row0 = wid * _RPW

    pltpu.async_copy(x_hbm.at[pl.ds(row0, _ROWS)], xb0, is0)

    pltpu.sync_copy(logits_hbm, lg)
    pltpu.sync_copy(alpha_hbm, av)

    def coef_body(cb, carry):
        sl = pl.ds(cb * _L, _L)
        l0, l1, l2, l3 = lg[0, sl], lg[1, sl], lg[2, sl], lg[3, sl]
        m = jnp.maximum(jnp.maximum(l0, l1), jnp.maximum(l2, l3))
        e0 = jnp.exp(l0 - m)
        e1 = jnp.exp(l1 - m)
        e2 = jnp.exp(l2 - m)
        e3 = jnp.exp(l3 - m)
        r = av[sl] / (e0 + e1 + e2 + e3)
        cf[0, sl] = e0 * r
        cf[1, sl] = e1 * r
        cf[2, sl] = e2 * r
        cf[3, sl] = e3 * r
        return carry

    lax.fori_loop(0, _CB, coef_body, 0)

    def _in_slice(k):
        return x_hbm.at[pl.ds(row0 + k * _ROWS, _ROWS)]

    def _out_slice(k):
        return out_hbm.at[pl.ds(row0 + k * _ROWS, _ROWS)]

    def compute(xbuf, obuf):
        @plsc.parallel_loop(0, _CB, step=1, unroll=1)
        def cb_body(cb):
            sl = pl.ds(cb * _L, _L)
            c0, c1, cq, cs = cf[0, sl], cf[1, sl], cf[2, sl], cf[3, sl]
            for r in range(_ROWS):
                x = xbuf[r, sl]
                u = jnp.minimum(jnp.exp(-x), 1e9)
                u2 = u * u
                a1 = 1.0 + u
                a2 = 1.0 + u2
                d = 1.0 / (a1 * a2)
                num = c1 * a1 * (1.0 - u2) + cs * a2
                obuf[r, sl] = x * (c0 + cq * x) + num * d

    def pair_body(p, carry):
        k0 = 2 * p
        k1 = k0 + 1
        pltpu.async_copy(_in_slice(k1), xb1, is1)
        pltpu.make_async_copy(_in_slice(k0), xb0, is0).wait()

        @pl.when(p >= 1)
        def _():
            pltpu.make_async_copy(ob0, _out_slice(k0 - 2), os0).wait()

        compute(xb0, ob0)
        pltpu.async_copy(ob0, _out_slice(k0), os0)

        @pl.when(p + 1 < _NCHUNK // 2)
        def _():
            pltpu.async_copy(_in_slice(k0 + 2), xb0, is0)

        pltpu.make_async_copy(_in_slice(k1), xb1, is1).wait()

        @pl.when(p >= 1)
        def _():
            pltpu.make_async_copy(ob1, _out_slice(k1 - 2), os1).wait()

        compute(xb1, ob1)
        pltpu.async_copy(ob1, _out_slice(k1), os1)
        return carry

    lax.fori_loop(0, _NCHUNK // 2, pair_body, 0)
    pltpu.make_async_copy(ob0, _out_slice(_NCHUNK - 2), os0).wait()
    pltpu.make_async_copy(ob1, _out_slice(_NCHUNK - 1), os1).wait()


def kernel(X, alpha, tf_prob_logits):
    n, f = X.shape
    logits_t = tf_prob_logits.T

    mesh = plsc.VectorSubcoreMesh(core_axis_name="c", subcore_axis_name="s")
    run = pl.kernel(
        _sc_body,
        mesh=mesh,
        out_type=jax.ShapeDtypeStruct((n, f), X.dtype),
        scratch_types=[
            pltpu.VMEM((_T, _F), jnp.float32),
            pltpu.VMEM((_F,), jnp.float32),
            pltpu.VMEM((_T, _F), jnp.float32),
            pltpu.VMEM((_ROWS, _F), jnp.float32),
            pltpu.VMEM((_ROWS, _F), jnp.float32),
            pltpu.VMEM((_ROWS, _F), jnp.float32),
            pltpu.VMEM((_ROWS, _F), jnp.float32),
            pltpu.SemaphoreType.DMA,
            pltpu.SemaphoreType.DMA,
            pltpu.SemaphoreType.DMA,
            pltpu.SemaphoreType.DMA,
        ],
    )
    return run(X, alpha, logits_t)
